# Initial kernel scaffold; baseline (speedup 1.0000x reference)
import jax
import jax.numpy as jnp
from jax.experimental import pallas as pl


def kernel(x, edge_index, batch, user_state, num_hop, params):
    raise NotImplementedError("write your pallas kernel here")


# speedup vs baseline: 4.0975x; 4.0975x over previous
import functools

import jax
import jax.numpy as jnp
from jax.experimental import pallas as pl

N = 10000
E = 320000
B = 128
IN = 128
HID = 256
OUT = 128
NC = 4
MAX_HOP = 8


def _hop_chain(u, num_hop, p):
    alpha = jax.nn.sigmoid(p['raw_alpha'])
    beta = jax.nn.sigmoid(p['raw_beta'])
    u0 = u[:, None]
    U_ = u0 @ p['Wu0'] + p['bu0']
    S_ = jnp.zeros_like(u0) @ p['Ws0'] + p['bs0']
    D_ = jnp.zeros_like(u0) @ p['Wd0'] + p['bd0']
    Ul, Sl, Dl = [], [], []
    for _ in range(MAX_HOP):
        U_ = U_ - alpha * U_ - beta * U_
        U_ = U_ @ p['Wu'] + p['bu']
        S_ = (S_ + alpha * U_) @ p['Ws'] + p['bs']
        D_ = (D_ + beta * U_) @ p['Wd'] + p['bd']
        Ul.append(U_); Sl.append(S_); Dl.append(D_)
    U = jnp.stack(Ul, axis=1)
    S = jnp.stack(Sl, axis=1)
    D = jnp.stack(Dl, axis=1)
    hop = jnp.clip(num_hop.astype(jnp.int32) - 1, 0, MAX_HOP - 1)
    bidx = jnp.arange(B)
    xg = jnp.concatenate([U[bidx, hop], S[bidx, hop], D[bidx, hop]], axis=1) @ p['Wx'] + p['bx']
    Uo = U @ p['lu'] + p['blu']
    So = S @ p['ls'] + p['bls']
    Do = D @ p['ld'] + p['bld']
    return xg, Uo, So, Do


def _direction(x, g_idx, s_idx, onehot, first, nonempty, x_first_relu, W1, b1, W2, b2):
    deg = jnp.ones((N,), jnp.float32).at[s_idx].add(1.0)
    dis = jax.lax.rsqrt(deg)
    xs = x * dis[:, None]
    A1 = xs + jnp.zeros_like(xs).at[s_idx].add(xs[g_idx])
    out1 = (A1 * dis[:, None]) @ W1 + b1
    q = jax.nn.relu(out1) @ W2[:HID] + onehot @ (x_first_relu @ W2[HID:])
    qs = q * dis[:, None]
    A2 = qs + jnp.zeros_like(qs).at[s_idx].add(qs[g_idx])
    out2 = jax.nn.relu(A2 * dis[:, None] + b2)
    cnt = jnp.sum(onehot, axis=0)
    pool_m = (onehot.T @ out2) / jnp.maximum(cnt, 1.0)[:, None]
    root_p = out1[first] * nonempty[:, None]
    return jnp.concatenate([pool_m, root_p], axis=1)


def kernel(x, edge_index, batch, user_state, num_hop, params):
    p = params
    u = jnp.sum(user_state, axis=(1, 2))
    xg, Uo, So, Do = _hop_chain(u, num_hop, p)

    onehot = (batch[:, None] == jnp.arange(B, dtype=batch.dtype)[None, :]).astype(jnp.float32)
    cnt = jnp.sum(onehot, axis=0)
    first = jnp.clip(jnp.cumsum(cnt).astype(jnp.int32) - cnt.astype(jnp.int32), 0, N - 1)
    nonempty = (cnt > 0).astype(jnp.float32)
    x_first_relu = jax.nn.relu(x[first])

    src, dst = edge_index[0], edge_index[1]
    TD = _direction(x, src, dst, onehot, first, nonempty, x_first_relu,
                    p['td_W1'], p['td_b1'], p['td_W2'], p['td_b2']) + xg
    BU = _direction(x, dst, src, onehot, first, nonempty, x_first_relu,
                    p['bu_W1'], p['bu_b1'], p['bu_W2'], p['bu_b2']) + xg

    logits = jnp.concatenate([BU, TD], axis=1) @ p['fcW'] + p['fcb']
    out = jax.nn.log_softmax(logits, axis=-1)
    return (out, Uo, So, Do)


# speedup vs baseline: 11.8685x; 2.8965x over previous
import functools

import jax
import jax.numpy as jnp
from jax import lax
from jax.experimental import pallas as pl
from jax.experimental.pallas import tpu as pltpu
from jax.experimental.pallas import tpu_sc as plsc

N = 10000
E = 320000
B = 128
IN = 128
HID = 256
OUT = 128
NC = 4
MAX_HOP = 8

NT = 16
EPT = E // NT
KP = 80
NCHUNK = EPT // KP
NP = 10240
NROW = NP // NT

_mesh = plsc.VectorSubcoreMesh(core_axis_name="c", subcore_axis_name="s")


@functools.partial(
    pl.kernel, mesh=_mesh,
    out_type=jax.ShapeDtypeStruct((2 * NP, IN), jnp.float32),
    scratch_types=[
        pltpu.VMEM((2, KP), jnp.int32),
        pltpu.VMEM((KP, IN), jnp.float32),
        pltpu.VMEM((8, 80), jnp.int32),
        pltpu.VMEM_SHARED((NP, IN), jnp.float32),
    ],
)
def _deg_kernel(edge_hbm, out_hbm, idx_v, buf_v, ii_v, deg_sh):
    c = lax.axis_index("c")
    s = lax.axis_index("s")
    iota = lax.iota(jnp.int32, 16)
    zeros16 = jnp.zeros((16,), jnp.float32)
    ones16 = jnp.ones((16,), jnp.float32)
    r0 = s * NROW
    for r in range(8):
        for k in range(5):
            ii_v[r, pl.ds(k * 16, 16)] = iota + (r0 + 80 * r + 16 * k)
    for i in range(KP):
        for k in range(IN // 16):
            buf_v[i, pl.ds(16 * k, 16)] = zeros16
    for r in range(8):
        pltpu.sync_copy(buf_v, deg_sh.at[ii_v.at[r]])
    plsc.subcore_barrier()

    for i in range(KP):
        for k in range(IN // 16):
            buf_v[i, pl.ds(16 * k, 16)] = ones16

    def _chunk(j, carry):
        base = (1 - c) * E + s * EPT + j * KP
        pltpu.sync_copy(edge_hbm.at[pl.ds(base, KP)], idx_v.at[0])
        pltpu.sync_copy(buf_v, deg_sh.at[idx_v.at[0]], add=True)
        return carry
    lax.fori_loop(0, NCHUNK, _chunk, 0)

    plsc.subcore_barrier()
    for r in range(8):
        pltpu.sync_copy(deg_sh.at[ii_v.at[r]], buf_v)
        pltpu.sync_copy(buf_v, out_hbm.at[pl.ds(c * NP + r0 + 80 * r, 80)])


@functools.partial(
    pl.kernel, mesh=_mesh,
    out_type=jax.ShapeDtypeStruct((2 * NP, IN), jnp.float32),
    scratch_types=[
        pltpu.VMEM((KP,), jnp.int32),
        pltpu.VMEM((2, KP), jnp.int32),
        pltpu.VMEM((KP, IN), jnp.float32),
        pltpu.VMEM((8, 80), jnp.int32),
        pltpu.VMEM_SHARED((NP, IN), jnp.float32),
    ],
)
def _prop_kernel(table_hbm, garr_hbm, sarr_hbm, out_hbm, gi_v, si_v, rows_v, ii_v, acc_sh):
    c = lax.axis_index("c")
    s = lax.axis_index("s")
    iota = lax.iota(jnp.int32, 16)
    r0 = s * NROW
    for r in range(8):
        for k in range(5):
            ii_v[r, pl.ds(k * 16, 16)] = iota + (r0 + 80 * r + 16 * k)
    for r in range(8):
        pltpu.sync_copy(table_hbm.at[pl.ds(c * NP + r0 + 80 * r, 80)],
                        rows_v.at[pl.ds(0, 80)])
        pltpu.sync_copy(rows_v.at[pl.ds(0, 80)], acc_sh.at[ii_v.at[r]])
    plsc.subcore_barrier()

    def _chunk(j, carry):
        base = c * E + s * EPT + j * KP
        pltpu.sync_copy(garr_hbm.at[pl.ds(base, KP)], gi_v)
        pltpu.sync_copy(sarr_hbm.at[pl.ds(base, KP)], si_v.at[0])
        pltpu.sync_copy(table_hbm.at[gi_v], rows_v)
        pltpu.sync_copy(rows_v, acc_sh.at[si_v.at[0]], add=True)
        return carry
    lax.fori_loop(0, NCHUNK, _chunk, 0)

    plsc.subcore_barrier()
    for r in range(8):
        pltpu.sync_copy(acc_sh.at[ii_v.at[r]], rows_v.at[pl.ds(0, 80)])
        pltpu.sync_copy(rows_v.at[pl.ds(0, 80)],
                        out_hbm.at[pl.ds(c * NP + r0 + 80 * r, 80)])


def _hop_chain(u, num_hop, p):
    alpha = jax.nn.sigmoid(p['raw_alpha'])
    beta = jax.nn.sigmoid(p['raw_beta'])
    u0 = u[:, None]
    U_ = u0 @ p['Wu0'] + p['bu0']
    S_ = jnp.zeros_like(u0) @ p['Ws0'] + p['bs0']
    D_ = jnp.zeros_like(u0) @ p['Wd0'] + p['bd0']
    Ul, Sl, Dl = [], [], []
    for _ in range(MAX_HOP):
        U_ = U_ - alpha * U_ - beta * U_
        U_ = U_ @ p['Wu'] + p['bu']
        S_ = (S_ + alpha * U_) @ p['Ws'] + p['bs']
        D_ = (D_ + beta * U_) @ p['Wd'] + p['bd']
        Ul.append(U_); Sl.append(S_); Dl.append(D_)
    U = jnp.stack(Ul, axis=1)
    S = jnp.stack(Sl, axis=1)
    D = jnp.stack(Dl, axis=1)
    hop = jnp.clip(num_hop.astype(jnp.int32) - 1, 0, MAX_HOP - 1)
    bidx = jnp.arange(B)
    xg = jnp.concatenate([U[bidx, hop], S[bidx, hop], D[bidx, hop]], axis=1) @ p['Wx'] + p['bx']
    Uo = U @ p['lu'] + p['blu']
    So = S @ p['ls'] + p['bls']
    Do = D @ p['ld'] + p['bld']
    return xg, Uo, So, Do


def kernel(x, edge_index, batch, user_state, num_hop, params):
    p = params
    u = jnp.sum(user_state, axis=(1, 2))
    xg, Uo, So, Do = _hop_chain(u, num_hop, p)

    onehot = (batch[:, None] == jnp.arange(B, dtype=batch.dtype)[None, :]).astype(jnp.float32)
    cnt = jnp.sum(onehot, axis=0)
    first = jnp.clip(jnp.cumsum(cnt).astype(jnp.int32) - cnt.astype(jnp.int32), 0, N - 1)
    nonempty = (cnt > 0).astype(jnp.float32)
    x_first_relu = jax.nn.relu(x[first])

    edge_flat = edge_index.reshape(-1)
    degs = _deg_kernel(edge_flat)
    deg_td = degs[:N, 0] + 1.0
    deg_bu = degs[NP:NP + N, 0] + 1.0
    dis_td = lax.rsqrt(deg_td)[:, None]
    dis_bu = lax.rsqrt(deg_bu)[:, None]

    src, dst = edge_index[0], edge_index[1]
    garr = jnp.concatenate([src, dst + NP])
    sarr = jnp.concatenate([dst, src])
    padrows = jnp.zeros((NP - N, IN), jnp.float32)
    tab1 = jnp.concatenate([x * dis_td, padrows, x * dis_bu, padrows], axis=0)
    A1 = _prop_kernel(tab1, garr, sarr)
    out1_td = (A1[:N] * dis_td) @ p['td_W1'] + p['td_b1']
    out1_bu = (A1[NP:NP + N] * dis_bu) @ p['bu_W1'] + p['bu_b1']

    q_td = jax.nn.relu(out1_td) @ p['td_W2'][:HID] + onehot @ (x_first_relu @ p['td_W2'][HID:])
    q_bu = jax.nn.relu(out1_bu) @ p['bu_W2'][:HID] + onehot @ (x_first_relu @ p['bu_W2'][HID:])

    tab2 = jnp.concatenate([q_td * dis_td, padrows, q_bu * dis_bu, padrows], axis=0)
    A2 = _prop_kernel(tab2, garr, sarr)
    out2_td = jax.nn.relu(A2[:N] * dis_td + p['td_b2'])
    out2_bu = jax.nn.relu(A2[NP:NP + N] * dis_bu + p['bu_b2'])

    inv_cnt = 1.0 / jnp.maximum(cnt, 1.0)[:, None]
    TD = jnp.concatenate([(onehot.T @ out2_td) * inv_cnt,
                          out1_td[first] * nonempty[:, None]], axis=1) + xg
    BU = jnp.concatenate([(onehot.T @ out2_bu) * inv_cnt,
                          out1_bu[first] * nonempty[:, None]], axis=1) + xg

    logits = jnp.concatenate([BU, TD], axis=1) @ p['fcW'] + p['fcb']
    return (jax.nn.log_softmax(logits, axis=-1), Uo, So, Do)


# speedup vs baseline: 24.1330x; 2.0334x over previous
import functools

import jax
import jax.numpy as jnp
from jax import lax
from jax.experimental import pallas as pl
from jax.experimental.pallas import tpu as pltpu
from jax.experimental.pallas import tpu_sc as plsc

N = 10000
E = 320000
B = 128
IN = 128
HID = 256
OUT = 128
NC = 4
MAX_HOP = 8

NT = 16
EPT = E // NT
KP = 80
NW = 10
NSB = EPT // (KP * NW)
NP = 10240
NROW = NP // NT

_mesh = plsc.VectorSubcoreMesh(core_axis_name="c", subcore_axis_name="s")


@functools.partial(
    pl.kernel, mesh=_mesh,
    out_type=jax.ShapeDtypeStruct((2 * NP, IN), jnp.float32),
    scratch_types=[
        pltpu.VMEM((NW, KP), jnp.int32),
        pltpu.VMEM((KP, IN), jnp.float32),
        pltpu.VMEM((8, 80), jnp.int32),
        pltpu.VMEM_SHARED((NP, IN), jnp.float32),
        pltpu.SemaphoreType.DMA,
        pltpu.SemaphoreType.DMA,
    ],
)
def _deg_kernel(edge_hbm, out_hbm, idx_v, buf_v, ii_v, deg_sh, isem, ssem):
    c = lax.axis_index("c")
    s = lax.axis_index("s")
    iota = lax.iota(jnp.int32, 16)
    zeros16 = jnp.zeros((16,), jnp.float32)
    ones16 = jnp.ones((16,), jnp.float32)
    r0 = s * NROW
    for r in range(8):
        for k in range(5):
            ii_v[r, pl.ds(k * 16, 16)] = iota + (r0 + 80 * r + 16 * k)
    for i in range(KP):
        for k in range(IN // 16):
            buf_v[i, pl.ds(16 * k, 16)] = zeros16
    for r in range(8):
        pltpu.sync_copy(buf_v, deg_sh.at[ii_v.at[r]])
    plsc.subcore_barrier()

    for i in range(KP):
        for k in range(IN // 16):
            buf_v[i, pl.ds(16 * k, 16)] = ones16

    def _chunk(j, carry):
        base = (1 - c) * E + s * EPT + j * (KP * NW)
        hi = [pltpu.async_copy(edge_hbm.at[pl.ds(base + w * KP, KP)],
                               idx_v.at[w], isem) for w in range(NW)]
        hs = []
        for w in range(NW):
            hi[w].wait()
            hs.append(pltpu.async_copy(buf_v, deg_sh.at[idx_v.at[w]],
                                       ssem, add=True))
        for h in hs:
            h.wait()
        return carry
    lax.fori_loop(0, NSB, _chunk, 0)

    plsc.subcore_barrier()
    for r in range(8):
        pltpu.sync_copy(deg_sh.at[ii_v.at[r]], buf_v)
        pltpu.sync_copy(buf_v, out_hbm.at[pl.ds(c * NP + r0 + 80 * r, 80)])


@functools.partial(
    pl.kernel, mesh=_mesh,
    out_type=jax.ShapeDtypeStruct((2 * NP, IN), jnp.float32),
    scratch_types=[
        pltpu.VMEM((NW * KP,), jnp.int32),
        pltpu.VMEM((NW, KP), jnp.int32),
        pltpu.VMEM((2, KP, IN), jnp.float32),
        pltpu.VMEM((8, 80), jnp.int32),
        pltpu.VMEM_SHARED((NP, IN), jnp.float32),
        pltpu.SemaphoreType.DMA,
        pltpu.SemaphoreType.DMA,
        pltpu.SemaphoreType.DMA,
        pltpu.SemaphoreType.DMA,
        pltpu.SemaphoreType.DMA,
    ],
)
def _prop_kernel(table_hbm, garr_hbm, sarr_hbm, out_hbm, gi_v, si_v, rows_v,
                 ii_v, acc_sh, isem, gsem0, gsem1, ssem0, ssem1):
    c = lax.axis_index("c")
    s = lax.axis_index("s")
    iota = lax.iota(jnp.int32, 16)
    r0 = s * NROW
    for r in range(8):
        for k in range(5):
            ii_v[r, pl.ds(k * 16, 16)] = iota + (r0 + 80 * r + 16 * k)
    for r in range(8):
        pltpu.sync_copy(table_hbm.at[pl.ds(c * NP + r0 + 80 * r, 80)],
                        rows_v.at[0])
        pltpu.sync_copy(rows_v.at[0], acc_sh.at[ii_v.at[r]])
    plsc.subcore_barrier()

    gsem = (gsem0, gsem1)
    ssem = (ssem0, ssem1)

    def _sblock(j, carry):
        base = c * E + s * EPT + j * (KP * NW)
        hgi = pltpu.async_copy(garr_hbm.at[pl.ds(base, KP * NW)], gi_v, isem)
        hsi = [pltpu.async_copy(sarr_hbm.at[pl.ds(base + w * KP, KP)],
                                si_v.at[w], isem) for w in range(NW)]
        hgi.wait()
        for h in hsi:
            h.wait()
        hg = {}
        hs = {}
        hg[0] = pltpu.async_copy(table_hbm.at[gi_v.at[pl.ds(0, KP)]],
                                 rows_v.at[0], gsem[0])
        for w in range(NW):
            b = w & 1
            if w + 1 < NW:
                nb = 1 - b
                if w >= 1:
                    hs[w - 1].wait()
                hg[w + 1] = pltpu.async_copy(
                    table_hbm.at[gi_v.at[pl.ds((w + 1) * KP, KP)]],
                    rows_v.at[nb], gsem[nb])
            hg[w].wait()
            hs[w] = pltpu.async_copy(rows_v.at[b], acc_sh.at[si_v.at[w]],
                                     ssem[b], add=True)
        hs[NW - 2].wait()
        hs[NW - 1].wait()
        return carry
    lax.fori_loop(0, NSB, _sblock, 0)

    plsc.subcore_barrier()
    for r in range(8):
        pltpu.sync_copy(acc_sh.at[ii_v.at[r]], rows_v.at[0])
        pltpu.sync_copy(rows_v.at[0],
                        out_hbm.at[pl.ds(c * NP + r0 + 80 * r, 80)])


def _hop_chain(u, num_hop, p):
    alpha = jax.nn.sigmoid(p['raw_alpha'])
    beta = jax.nn.sigmoid(p['raw_beta'])
    u0 = u[:, None]
    U_ = u0 @ p['Wu0'] + p['bu0']
    S_ = jnp.zeros_like(u0) @ p['Ws0'] + p['bs0']
    D_ = jnp.zeros_like(u0) @ p['Wd0'] + p['bd0']
    Ul, Sl, Dl = [], [], []
    for _ in range(MAX_HOP):
        U_ = U_ - alpha * U_ - beta * U_
        U_ = U_ @ p['Wu'] + p['bu']
        S_ = (S_ + alpha * U_) @ p['Ws'] + p['bs']
        D_ = (D_ + beta * U_) @ p['Wd'] + p['bd']
        Ul.append(U_); Sl.append(S_); Dl.append(D_)
    U = jnp.stack(Ul, axis=1)
    S = jnp.stack(Sl, axis=1)
    D = jnp.stack(Dl, axis=1)
    hop = jnp.clip(num_hop.astype(jnp.int32) - 1, 0, MAX_HOP - 1)
    bidx = jnp.arange(B)
    xg = jnp.concatenate([U[bidx, hop], S[bidx, hop], D[bidx, hop]], axis=1) @ p['Wx'] + p['bx']
    Uo = U @ p['lu'] + p['blu']
    So = S @ p['ls'] + p['bls']
    Do = D @ p['ld'] + p['bld']
    return xg, Uo, So, Do


def kernel(x, edge_index, batch, user_state, num_hop, params):
    p = params
    u = jnp.sum(user_state, axis=(1, 2))
    xg, Uo, So, Do = _hop_chain(u, num_hop, p)

    onehot = (batch[:, None] == jnp.arange(B, dtype=batch.dtype)[None, :]).astype(jnp.float32)
    cnt = jnp.sum(onehot, axis=0)
    first = jnp.clip(jnp.cumsum(cnt).astype(jnp.int32) - cnt.astype(jnp.int32), 0, N - 1)
    nonempty = (cnt > 0).astype(jnp.float32)
    x_first_relu = jax.nn.relu(x[first])

    edge_flat = edge_index.reshape(-1)
    degs = _deg_kernel(edge_flat)
    deg_td = degs[:N, 0] + 1.0
    deg_bu = degs[NP:NP + N, 0] + 1.0
    dis_td = lax.rsqrt(deg_td)[:, None]
    dis_bu = lax.rsqrt(deg_bu)[:, None]

    src, dst = edge_index[0], edge_index[1]
    garr = jnp.concatenate([src, dst + NP])
    sarr = jnp.concatenate([dst, src])
    padrows = jnp.zeros((NP - N, IN), jnp.float32)
    tab1 = jnp.concatenate([x * dis_td, padrows, x * dis_bu, padrows], axis=0)
    A1 = _prop_kernel(tab1, garr, sarr)
    out1_td = (A1[:N] * dis_td) @ p['td_W1'] + p['td_b1']
    out1_bu = (A1[NP:NP + N] * dis_bu) @ p['bu_W1'] + p['bu_b1']

    q_td = jax.nn.relu(out1_td) @ p['td_W2'][:HID] + onehot @ (x_first_relu @ p['td_W2'][HID:])
    q_bu = jax.nn.relu(out1_bu) @ p['bu_W2'][:HID] + onehot @ (x_first_relu @ p['bu_W2'][HID:])

    tab2 = jnp.concatenate([q_td * dis_td, padrows, q_bu * dis_bu, padrows], axis=0)
    A2 = _prop_kernel(tab2, garr, sarr)
    out2_td = jax.nn.relu(A2[:N] * dis_td + p['td_b2'])
    out2_bu = jax.nn.relu(A2[NP:NP + N] * dis_bu + p['bu_b2'])

    inv_cnt = 1.0 / jnp.maximum(cnt, 1.0)[:, None]
    TD = jnp.concatenate([(onehot.T @ out2_td) * inv_cnt,
                          out1_td[first] * nonempty[:, None]], axis=1) + xg
    BU = jnp.concatenate([(onehot.T @ out2_bu) * inv_cnt,
                          out1_bu[first] * nonempty[:, None]], axis=1) + xg

    logits = jnp.concatenate([BU, TD], axis=1) @ p['fcW'] + p['fcb']
    return (jax.nn.log_softmax(logits, axis=-1), Uo, So, Do)
